# baseline (speedup 1.0000x reference)
import functools

import jax
import jax.numpy as jnp
from jax import lax
from jax.experimental import pallas as pl
from jax.experimental.pallas import tpu as pltpu
from jax.experimental.pallas import tpu_sc as plsc

_info = plsc.get_sparse_core_info()
_NC = _info.num_cores
_NS = _info.num_subcores
_NW = _NC * _NS

_CHUNK = 32
_NBUF = 3


@functools.cache
def _make_gather(bsz, s, v, d):
    n = bsz * s
    per_w = n // _NW
    wpb = _NW // bsz
    n_chunks = per_w // _CHUNK
    n_groups = -(-n_chunks // _NBUF)
    assert per_w * _NW == n and n_chunks * _CHUNK == per_w
    assert wpb * bsz == _NW and n_chunks >= _NBUF
    mesh = plsc.VectorSubcoreMesh(core_axis_name="c", subcore_axis_name="s")

    @functools.partial(
        pl.kernel,
        mesh=mesh,
        out_type=jax.ShapeDtypeStruct((bsz, s, d), jnp.float32),
        scratch_types=[
            pltpu.VMEM((per_w,), jnp.int32),
            pltpu.VMEM((_NBUF, _CHUNK, 1, d), jnp.float32),
        ]
        + [pltpu.SemaphoreType.DMA] * (2 * _NBUF),
    )
    def gather(x_hbm, pe_hbm, out_hbm, idx_v, rows_v, *sems):
        gsems = sems[:_NBUF]
        osems = sems[_NBUF:]
        wid = lax.axis_index("s") * _NC + lax.axis_index("c")
        brow = wid // wpb
        base = (wid % wpb) * per_w
        pltpu.sync_copy(x_hbm.at[brow, pl.ds(base, per_w)], idx_v)

        def g_start(c, b):
            pltpu.async_copy(
                pe_hbm.at[idx_v.at[pl.ds(c * _CHUNK, _CHUNK)]],
                rows_v.at[b],
                gsems[b],
            )

        def g_wait(c, b):
            pltpu.make_async_copy(
                pe_hbm.at[idx_v.at[pl.ds(c * _CHUNK, _CHUNK)]],
                rows_v.at[b],
                gsems[b],
            ).wait()

        def o_start(c, b):
            pltpu.async_copy(
                rows_v.at[b, :, 0],
                out_hbm.at[brow, pl.ds(base + c * _CHUNK, _CHUNK)],
                osems[b],
            )

        def o_wait(c, b):
            pltpu.make_async_copy(
                rows_v.at[b, :, 0],
                out_hbm.at[brow, pl.ds(base + c * _CHUNK, _CHUNK)],
                osems[b],
            ).wait()

        for b in range(_NBUF):
            g_start(b, b)

        def body(g, carry):
            for b in range(_NBUF):
                c = g * _NBUF + b

                @pl.when(c < n_chunks)
                def _():
                    g_wait(c, b)
                    o_start(c, b)
                    o_wait(c, b)

                @pl.when(c + _NBUF < n_chunks)
                def _():
                    g_start(c + _NBUF, b)

            return carry

        lax.fori_loop(0, n_groups, body, 0)

    return gather


def kernel(x, pe):
    b, s = x.shape
    v, _, d = pe.shape
    return _make_gather(b, s, v, d)(x.astype(jnp.int32), pe)


# speedup vs baseline: 1.0084x; 1.0084x over previous
import functools

import jax
import jax.numpy as jnp
from jax import lax
from jax.experimental import pallas as pl
from jax.experimental.pallas import tpu as pltpu
from jax.experimental.pallas import tpu_sc as plsc

_info = plsc.get_sparse_core_info()
_NC = _info.num_cores
_NS = _info.num_subcores
_NW = _NC * _NS

_CHUNK = 16
_NBUF = 6


@functools.cache
def _make_gather(bsz, s, v, d):
    n = bsz * s
    per_w = n // _NW
    wpb = _NW // bsz
    n_chunks = per_w // _CHUNK
    n_groups = -(-n_chunks // _NBUF)
    assert per_w * _NW == n and n_chunks * _CHUNK == per_w
    assert wpb * bsz == _NW and n_chunks >= _NBUF
    mesh = plsc.VectorSubcoreMesh(core_axis_name="c", subcore_axis_name="s")

    @functools.partial(
        pl.kernel,
        mesh=mesh,
        out_type=jax.ShapeDtypeStruct((bsz, s, d), jnp.float32),
        scratch_types=[
            pltpu.VMEM((per_w,), jnp.int32),
            pltpu.VMEM((_NBUF, _CHUNK, 1, d), jnp.float32),
        ]
        + [pltpu.SemaphoreType.DMA] * (2 * _NBUF),
    )
    def gather(x_hbm, pe_hbm, out_hbm, idx_v, rows_v, *sems):
        gsems = sems[:_NBUF]
        osems = sems[_NBUF:]
        wid = lax.axis_index("s") * _NC + lax.axis_index("c")
        brow = wid // wpb
        base = (wid % wpb) * per_w
        pltpu.sync_copy(x_hbm.at[brow, pl.ds(base, per_w)], idx_v)

        def g_start(c, b):
            pltpu.async_copy(
                pe_hbm.at[idx_v.at[pl.ds(c * _CHUNK, _CHUNK)]],
                rows_v.at[b],
                gsems[b],
            )

        def g_wait(c, b):
            pltpu.make_async_copy(
                pe_hbm.at[idx_v.at[pl.ds(c * _CHUNK, _CHUNK)]],
                rows_v.at[b],
                gsems[b],
            ).wait()

        def o_start(c, b):
            pltpu.async_copy(
                rows_v.at[b, :, 0],
                out_hbm.at[brow, pl.ds(base + c * _CHUNK, _CHUNK)],
                osems[b],
            )

        def o_wait(c, b):
            pltpu.make_async_copy(
                rows_v.at[b, :, 0],
                out_hbm.at[brow, pl.ds(base + c * _CHUNK, _CHUNK)],
                osems[b],
            ).wait()

        for b in range(_NBUF):
            g_start(b, b)

        def body(g, carry):
            for b in range(_NBUF):
                c = g * _NBUF + b

                @pl.when(c < n_chunks)
                def _():
                    g_wait(c, b)
                    o_start(c, b)
                    o_wait(c, b)

                @pl.when(c + _NBUF < n_chunks)
                def _():
                    g_start(c + _NBUF, b)

            return carry

        lax.fori_loop(0, n_groups, body, 0)

    return gather


def kernel(x, pe):
    b, s = x.shape
    v, _, d = pe.shape
    return _make_gather(b, s, v, d)(x.astype(jnp.int32), pe)
